# baseline (speedup 1.0000x reference)
import functools

import jax
import jax.numpy as jnp
from jax import lax
from jax.experimental import pallas as pl
from jax.experimental.pallas import tpu as pltpu
from jax.experimental.pallas import tpu_sc as plsc

_NUM_ACTION = 125
_NUM_NOUN = 352
_DBINS = 5
_BS = 64
_C = 64 + _NUM_ACTION + _NUM_NOUN
_SPATIAL = 256
_ROW = _C * _DBINS * _SPATIAL
_CONF_OFF = 63 * _DBINS * _SPATIAL
_CLS_OFF = 64 * _DBINS * _SPATIAL
_NCLS = _NUM_ACTION + _NUM_NOUN
_NPAD = 480
_NW = 32
_BPW = _BS // _NW


def _sc_gather_body(pred_hbm, offs_hbm, out_hbm, conf_v, offs_v, idx_v, rows_v,
                    red_f, red_i, sem):
    wid = lax.axis_index("c") * 16 + lax.axis_index("s")
    pltpu.sync_copy(offs_hbm, offs_v)
    lanes = lax.iota(jnp.int32, 16)
    for i in range(_BPW):
        b = wid * _BPW + i
        base = b * _ROW
        pltpu.sync_copy(
            pred_hbm.at[pl.ds(base + _CONF_OFF, _DBINS * _SPATIAL)], conf_v)

        def amax_body(j, carry):
            bv, bf = carry
            v = conf_v[pl.ds(j * 16, 16)]
            sig = 1.0 / (1.0 + jnp.exp(-v))
            f = 5 * lanes + (80 * (j % 16) + j // 16)
            better = (sig > bv) | ((sig == bv) & (f < bf))
            return jnp.where(better, sig, bv), jnp.where(better, f, bf)

        bv, bf = lax.fori_loop(
            0, (_DBINS * _SPATIAL) // 16, amax_body,
            (jnp.full((16,), -1.0, jnp.float32), jnp.zeros((16,), jnp.int32)))

        for k in (1, 2, 4, 8):
            red_f[...] = bv
            red_i[...] = bf
            perm = lanes ^ k
            ov = plsc.load_gather(red_f, [perm])
            of = plsc.load_gather(red_i, [perm])
            better = (ov > bv) | ((ov == bv) & (of < bf))
            bv = jnp.where(better, ov, bv)
            bf = jnp.where(better, of, bf)
        d = bf % _DBINS
        s = bf // _DBINS
        c0 = base + _SPATIAL * d + s

        def idx_body(k, carry):
            idx_v[pl.ds(k * 16, 16)] = offs_v[pl.ds(k * 16, 16)] + c0
            return carry

        lax.fori_loop(0, _NPAD // 16, idx_body, 0)

        copies = [
            pltpu.make_async_copy(
                pred_hbm.at[idx_v.at[pl.ds(g * 120, 120)]],
                rows_v.at[pl.ds(g * 120, 120)], sem)
            for g in range(4)
        ]
        for cp in copies:
            cp.start()
        for cp in copies:
            cp.wait()
        pltpu.sync_copy(rows_v, out_hbm.at[b])


@functools.cache
def _sc_gather():
    return pl.kernel(
        _sc_gather_body,
        out_type=jax.ShapeDtypeStruct((_BS, _NPAD), jnp.float32),
        mesh=plsc.VectorSubcoreMesh(
            core_axis_name="c", subcore_axis_name="s",
            num_cores=2, num_subcores=16),
        compiler_params=pltpu.CompilerParams(needs_layout_passes=False),
        scratch_types=[
            pltpu.VMEM((_DBINS * _SPATIAL,), jnp.float32),
            pltpu.VMEM((_NPAD,), jnp.int32),
            pltpu.VMEM((_NPAD,), jnp.int32),
            pltpu.VMEM((_NPAD,), jnp.float32),
            pltpu.VMEM((16,), jnp.float32),
            pltpu.VMEM((16,), jnp.int32),
            pltpu.SemaphoreType.DMA,
        ],
    )


def _ce_body(chosen_ref, ga_ref, gn_ref, out_ref):
    x = chosen_ref[...]
    lane = lax.broadcasted_iota(jnp.int32, (_BS, _NPAD), 1)
    mask_a = lane < _NUM_ACTION
    mask_n = (lane >= _NUM_ACTION) & (lane < _NCLS)
    neg = jnp.float32(-1e30)
    xa = jnp.where(mask_a, x, neg)
    xn = jnp.where(mask_n, x, neg)
    ma = jnp.max(xa, axis=1, keepdims=True)
    mn = jnp.max(xn, axis=1, keepdims=True)
    sa = jnp.sum(jnp.where(mask_a, jnp.exp(xa - ma), 0.0), axis=1, keepdims=True)
    sn = jnp.sum(jnp.where(mask_n, jnp.exp(xn - mn), 0.0), axis=1, keepdims=True)
    lse_a = ma + jnp.log(sa)
    lse_n = mn + jnp.log(sn)
    pa = jnp.sum(jnp.where(lane == ga_ref[...], x, 0.0), axis=1, keepdims=True)
    pn = jnp.sum(jnp.where(lane == gn_ref[...] + _NUM_ACTION, x, 0.0),
                 axis=1, keepdims=True)
    la = jnp.sum(lse_a - pa) * 0.5
    ln = jnp.sum(lse_n - pn) * 0.5
    out_ref[0] = la + ln
    out_ref[1] = la
    out_ref[2] = ln


_ce_call = pl.pallas_call(
    _ce_body,
    out_shape=jax.ShapeDtypeStruct((3,), jnp.float32),
    out_specs=pl.BlockSpec(memory_space=pltpu.MemorySpace.SMEM),
)


def kernel(pred, action_gt, noun_gt):
    pred_flat = pred.reshape(-1)
    e = jnp.arange(_NPAD, dtype=jnp.int32)
    offs = jnp.where(e < _NCLS, _CLS_OFF + _DBINS * _SPATIAL * e, 0)
    chosen = _sc_gather()(pred_flat, offs)
    ga = action_gt.astype(jnp.int32).reshape(_BS, 1)
    gn = noun_gt.astype(jnp.int32).reshape(_BS, 1)
    out = _ce_call(chosen, ga, gn)
    return (out[0], out[1], out[2])


# speedup vs baseline: 38.3296x; 38.3296x over previous
import functools

import jax
import jax.numpy as jnp
from jax import lax
from jax.experimental import pallas as pl
from jax.experimental.pallas import tpu as pltpu
from jax.experimental.pallas import tpu_sc as plsc

_NUM_ACTION = 125
_NUM_NOUN = 352
_DBINS = 5
_BS = 64
_C = 64 + _NUM_ACTION + _NUM_NOUN
_NCH = _C * _DBINS
_SPATIAL = 256
_CONF0 = 63 * _DBINS
_CLS0 = 64 * _DBINS
_NCLS = _NUM_ACTION + _NUM_NOUN
_NPAD = 480
_NW = 32
_BPW = _BS // _NW


def _sc_gather_body(pred_hbm, out_hbm, conf_v, cell_v, rows_v, red_f, red_i):
    wid = lax.axis_index("c") * 16 + lax.axis_index("s")
    lanes = lax.iota(jnp.int32, 16)
    for i in range(_BPW):
        b = wid * _BPW + i
        row0 = b * _SPATIAL
        pltpu.sync_copy(
            pred_hbm.at[pl.ds(row0, _SPATIAL), pl.ds(256, 128)], conf_v)

        def amax_body(j, carry):
            bv, bf = carry
            s_vec = 16 * (j % 16) + lanes
            col = jnp.broadcast_to(59 + j // 16, (16,))
            v = plsc.load_gather(conf_v, [s_vec, col])
            sig = 1.0 / (1.0 + jnp.exp(-v))
            f = 5 * s_vec + (j // 16)
            better = (sig > bv) | ((sig == bv) & (f < bf))
            return jnp.where(better, sig, bv), jnp.where(better, f, bf)

        bv, bf = lax.fori_loop(
            0, (_DBINS * _SPATIAL) // 16, amax_body,
            (jnp.full((16,), -1.0, jnp.float32), jnp.zeros((16,), jnp.int32)))

        for k in (1, 2, 4, 8):
            red_f[...] = bv
            red_i[...] = bf
            perm = lanes ^ k
            ov = plsc.load_gather(red_f, [perm])
            of = plsc.load_gather(red_i, [perm])
            better = (ov > bv) | ((ov == bv) & (of < bf))
            bv = jnp.where(better, ov, bv)
            bf = jnp.where(better, of, bf)
        fm = bf[0]
        d = fm % _DBINS
        s = fm // _DBINS

        pltpu.sync_copy(
            pred_hbm.at[pl.ds(row0 + (s // 8) * 8, 8), :], cell_v)
        srow = jnp.broadcast_to(s % 8, (16,))

        def sel_body(k, carry):
            idxc = _CLS0 + d + 5 * (16 * k + lanes)
            idxc = jnp.minimum(idxc, _NCH - 1)
            rows_v[pl.ds(k * 16, 16)] = plsc.load_gather(cell_v, [srow, idxc])
            return carry

        lax.fori_loop(0, _NPAD // 16, sel_body, 0)
        pltpu.sync_copy(rows_v, out_hbm.at[b])


@functools.cache
def _sc_gather():
    return pl.kernel(
        _sc_gather_body,
        out_type=jax.ShapeDtypeStruct((_BS, _NPAD), jnp.float32),
        mesh=plsc.VectorSubcoreMesh(
            core_axis_name="c", subcore_axis_name="s",
            num_cores=2, num_subcores=16),
        compiler_params=pltpu.CompilerParams(
            needs_layout_passes=False, use_tc_tiling_on_sc=True),
        scratch_types=[
            pltpu.VMEM((_SPATIAL, 128), jnp.float32),
            pltpu.VMEM((8, _NCH), jnp.float32),
            pltpu.VMEM((_NPAD,), jnp.float32),
            pltpu.VMEM((16,), jnp.float32),
            pltpu.VMEM((16,), jnp.int32),
        ],
    )


def _ce_body(chosen_ref, ga_ref, gn_ref, out_ref):
    x = chosen_ref[...]
    lane = lax.broadcasted_iota(jnp.int32, (_BS, _NPAD), 1)
    mask_a = lane < _NUM_ACTION
    mask_n = (lane >= _NUM_ACTION) & (lane < _NCLS)
    neg = jnp.float32(-1e30)
    xa = jnp.where(mask_a, x, neg)
    xn = jnp.where(mask_n, x, neg)
    ma = jnp.max(xa, axis=1, keepdims=True)
    mn = jnp.max(xn, axis=1, keepdims=True)
    sa = jnp.sum(jnp.where(mask_a, jnp.exp(xa - ma), 0.0), axis=1, keepdims=True)
    sn = jnp.sum(jnp.where(mask_n, jnp.exp(xn - mn), 0.0), axis=1, keepdims=True)
    lse_a = ma + jnp.log(sa)
    lse_n = mn + jnp.log(sn)
    pa = jnp.sum(jnp.where(lane == ga_ref[...], x, 0.0), axis=1, keepdims=True)
    pn = jnp.sum(jnp.where(lane == gn_ref[...] + _NUM_ACTION, x, 0.0),
                 axis=1, keepdims=True)
    la = jnp.sum(lse_a - pa) * 0.5
    ln = jnp.sum(lse_n - pn) * 0.5
    out_ref[0] = la + ln
    out_ref[1] = la
    out_ref[2] = ln


_ce_call = pl.pallas_call(
    _ce_body,
    out_shape=jax.ShapeDtypeStruct((3,), jnp.float32),
    out_specs=pl.BlockSpec(memory_space=pltpu.MemorySpace.SMEM),
)


def kernel(pred, action_gt, noun_gt):
    pred_m = jnp.transpose(pred, (0, 2, 3, 1)).reshape(_BS * _SPATIAL, _NCH)
    chosen = _sc_gather()(pred_m)
    ga = action_gt.astype(jnp.int32).reshape(_BS, 1)
    gn = noun_gt.astype(jnp.int32).reshape(_BS, 1)
    out = _ce_call(chosen, ga, gn)
    return (out[0], out[1], out[2])
